# baseline (speedup 1.0000x reference)
import functools

import jax
import jax.numpy as jnp
from jax import lax
from jax.experimental import pallas as pl
from jax.experimental.pallas import tpu as pltpu
from jax.experimental.pallas import tpu_sc as plsc

B = 64
N = 32768
C = 5
L = 16
NC, NS = 2, 16
NW = NC * NS
ROWS_PER_W = B // NW
ROW_F = N * C
NCHUNK = 4
CHUNK_F = ROW_F // NCHUNK
CAND_PER_CHUNK = CHUNK_F // C
ITERS = CAND_PER_CHUNK // L
EP_SIZE = B * ROW_F
BIG_IDX = 1 << 30

_mesh = plsc.VectorSubcoreMesh(core_axis_name="c", subcore_axis_name="s")


@functools.partial(
    pl.kernel,
    out_type=jax.ShapeDtypeStruct((B * 8,), jnp.float32),
    mesh=_mesh,
    scratch_types=[
        pltpu.VMEM((CHUNK_F,), jnp.float32),
        pltpu.VMEM((CHUNK_F,), jnp.float32),
        pltpu.VMEM((16,), jnp.float32),
        pltpu.VMEM((16,), jnp.float32),
        pltpu.VMEM((16,), jnp.float32),
        pltpu.VMEM((16,), jnp.float32),
        pltpu.SemaphoreType.DMA,
        pltpu.SemaphoreType.DMA,
        pltpu.SemaphoreType.DMA,
    ],
    compiler_params=pltpu.CompilerParams(needs_layout_passes=False),
)
def _select_kernel(ep_hbm, line_hbm, perp_hbm, out_hbm,
                   buf0, buf1, ebuf, lbuf, pbuf, obuf,
                   sem0, sem1, sem_s):
    wid = lax.axis_index("s") * NC + lax.axis_index("c")
    base_flat = pl.multiple_of(wid * (ROWS_PER_W * ROW_F), 8)

    iota = lax.iota(jnp.int32, L)
    iota5 = iota * 5

    bufs = (buf0, buf1)
    sems = (sem0, sem1)
    total_chunks = ROWS_PER_W * NCHUNK

    def chunk_scan(buf, cand_base, mx, ai):
        pos0 = cand_base + iota

        def body(i, carry):
            mx, ai = carry
            cv = plsc.load_gather(buf, [i * (L * 5) + iota5])
            m = cv > mx
            mx = jnp.where(m, cv, mx)
            ai = jnp.where(m, pos0 + i * L, ai)
            return mx, ai

        return lax.fori_loop(0, ITERS, body, (mx, ai))

    def finalize_row(r, mx, ai):
        row_max = jnp.max(mx)
        g = jnp.min(jnp.where(mx == row_max, ai,
                              jnp.int32(BIG_IDX)))

        o = (r * N + g) * 5
        e_base = pl.multiple_of(jnp.minimum((o >> 3) << 3, EP_SIZE - 16), 8)
        erel = o - e_base
        pltpu.async_copy(ep_hbm.at[pl.ds(e_base, 16)], ebuf, sem_s).wait()
        ev = plsc.load_gather(ebuf, [erel + 1 + (iota & 3)])
        plsc.store_scatter(obuf, [iota], ev, mask=iota < 4)

        lo = (r * N + g) * 2
        l_base = pl.multiple_of((lo >> 3) << 3, 8)
        lrel = lo - l_base
        pltpu.async_copy(line_hbm.at[pl.ds(l_base, 8)],
                         lbuf.at[pl.ds(0, 8)], sem_s).wait()
        lv = plsc.load_gather(lbuf, [lrel + (iota & 1)])
        plsc.store_scatter(obuf, [iota], lv,
                           mask=jnp.logical_and(iota >= 4, iota < 6))

        pltpu.async_copy(perp_hbm.at[pl.ds(l_base, 8)],
                         pbuf.at[pl.ds(0, 8)], sem_s).wait()
        pv = plsc.load_gather(pbuf, [lrel + (iota & 1)])
        plsc.store_scatter(obuf, [iota], pv,
                           mask=jnp.logical_and(iota >= 6, iota < 8))

        pltpu.async_copy(obuf.at[pl.ds(0, 8)],
                         out_hbm.at[pl.ds(pl.multiple_of(r * 8, 8), 8)],
                         sem_s).wait()

    copies = [None, None]
    copies[0] = pltpu.async_copy(
        ep_hbm.at[pl.ds(base_flat, CHUNK_F)], buf0, sem0)

    neg_inf = jnp.full((L,), -jnp.inf, jnp.float32)
    zeros_i = jnp.zeros((L,), jnp.int32)
    mx, ai = neg_inf, zeros_i

    for c in range(total_chunks):
        par = c & 1
        if c + 1 < total_chunks:
            copies[1 - par] = pltpu.async_copy(
                ep_hbm.at[pl.ds(base_flat + (c + 1) * CHUNK_F, CHUNK_F)],
                bufs[1 - par], sems[1 - par])
        copies[par].wait()
        cand_base = (c % NCHUNK) * CAND_PER_CHUNK
        mx, ai = chunk_scan(bufs[par], cand_base, mx, ai)
        if c % NCHUNK == NCHUNK - 1:
            r = wid * ROWS_PER_W + c // NCHUNK
            finalize_row(r, mx, ai)
            mx, ai = neg_inf, zeros_i


def kernel(endpoints_pred, line_vec, perp_vec):
    ep = endpoints_pred.reshape(-1)
    lv = line_vec.reshape(-1)
    pv = perp_vec.reshape(-1)
    out = _select_kernel(ep, lv, pv).reshape(B, 8)
    selected_endpoints = out[:, 0:4].reshape(B, 2, 2)
    selected_line_vec = out[:, 4:6]
    selected_perp_vec = out[:, 6:8]
    return (selected_endpoints, selected_line_vec, selected_perp_vec)


# speedup vs baseline: 162.1065x; 162.1065x over previous
import functools

import jax
import jax.numpy as jnp
from jax import lax
from jax.experimental import pallas as pl
from jax.experimental.pallas import tpu as pltpu
from jax.experimental.pallas import tpu_sc as plsc

B = 64
N = 32768
L = 16
NC, NS = 2, 16
QCOLS = N // 4
HCOLS = QCOLS // 2
ITERS = HCOLS // L
BIG_IDX = 1 << 30

_mesh = plsc.VectorSubcoreMesh(core_axis_name="c", subcore_axis_name="s")


@functools.partial(
    pl.kernel,
    out_type=(jax.ShapeDtypeStruct((B * 8,), jnp.float32),
              jax.ShapeDtypeStruct((2 * 16 * 16,), jnp.float32),
              jax.ShapeDtypeStruct((2 * 16 * 16,), jnp.int32)),
    mesh=_mesh,
    scratch_types=[
        pltpu.VMEM((8, HCOLS), jnp.float32),
        pltpu.VMEM((8, HCOLS), jnp.float32),
        pltpu.VMEM((16,), jnp.float32),
        pltpu.VMEM((16,), jnp.int32),
        pltpu.VMEM((64,), jnp.float32),
        pltpu.VMEM((64,), jnp.int32),
        pltpu.VMEM((4, 8, 128), jnp.float32),
        pltpu.VMEM((2, 128), jnp.float32),
        pltpu.VMEM((2, 128), jnp.float32),
        pltpu.VMEM((16,), jnp.float32),
        pltpu.SemaphoreType.DMA,
        pltpu.SemaphoreType.DMA,
        pltpu.SemaphoreType.DMA,
    ],
    compiler_params=pltpu.CompilerParams(needs_layout_passes=False),
)
def _select_kernel(ep_hbm, line_hbm, perp_hbm,
                   out_hbm, xval_hbm, xidx_hbm,
                   c0, c1, valbuf, idxbuf, qval, qidx,
                   ebuf, lbuf, pbuf, obuf,
                   sem0, sem1, sem_f):
    cid = lax.axis_index("c")
    sid = lax.axis_index("s")
    band = cid * 4 + (sid >> 2)
    k = sid & 3
    rb = pl.multiple_of(band * 8, 8)
    col0 = pl.multiple_of(k * QCOLS, 128)

    iota = lax.iota(jnp.int32, L)

    def row_scan(buf, s_r, cand_base, mx, ai):
        pos0 = cand_base + iota

        def body(i, carry):
            mx, ai = carry
            cv = buf[s_r, pl.ds(i * L, L)]
            m = cv > mx
            mx = jnp.where(m, cv, mx)
            ai = jnp.where(m, pos0 + i * L, ai)
            return mx, ai

        return lax.fori_loop(0, ITERS, body, (mx, ai), unroll=8)

    h0 = pltpu.async_copy(
        ep_hbm.at[0, pl.ds(rb, 8), pl.ds(col0, HCOLS)], c0, sem0)
    h1 = pltpu.async_copy(
        ep_hbm.at[0, pl.ds(rb, 8), pl.ds(col0 + HCOLS, HCOLS)], c1, sem1)

    neg_inf = jnp.full((L,), -jnp.inf, jnp.float32)
    zeros_i = jnp.zeros((L,), jnp.int32)
    accs = [(neg_inf, zeros_i)] * 8

    h0.wait()
    for s_r in range(8):
        accs[s_r] = row_scan(c0, s_r, col0, *accs[s_r])
    h1.wait()
    for s_r in range(8):
        accs[s_r] = row_scan(c1, s_r, col0 + HCOLS, *accs[s_r])

    valv = jnp.zeros((L,), jnp.float32)
    idxv = jnp.zeros((L,), jnp.int32)
    for s_r in range(8):
        mx, ai = accs[s_r]
        row_max = jnp.max(mx)
        g = jnp.min(jnp.where(mx == row_max, ai, jnp.int32(BIG_IDX)))
        valv = jnp.where(iota == s_r, row_max, valv)
        idxv = jnp.where(iota == s_r, g, idxv)
    valbuf[...] = valv
    idxbuf[...] = idxv

    slot = cid * 16 + sid
    pltpu.sync_copy(valbuf,
                    xval_hbm.at[pl.ds(pl.multiple_of(slot * 16, 8), 16)])
    pltpu.sync_copy(idxbuf,
                    xidx_hbm.at[pl.ds(pl.multiple_of(slot * 16, 8), 16)])
    plsc.subcore_barrier()
    bslot = cid * 16 + (sid & ~3)
    pltpu.sync_copy(xval_hbm.at[pl.ds(pl.multiple_of(bslot * 16, 8), 64)],
                    qval)
    pltpu.sync_copy(xidx_hbm.at[pl.ds(pl.multiple_of(bslot * 16, 8), 64)],
                    qidx)

    mv = qval[pl.ds(0, L)]
    mi = qidx[pl.ds(0, L)]
    for q in range(1, 4):
        v = qval[pl.ds(q * 16, L)]
        i_ = qidx[pl.ds(q * 16, L)]
        m = v > mv
        mv = jnp.where(m, v, mv)
        mi = jnp.where(m, i_, mi)

    for j in range(2):
        lane = k * 2 + j
        r = rb + lane
        g = jnp.max(jnp.where(iota == lane, mi, jnp.int32(0)))
        g = jnp.clip(g, 0, N - 1)
        g128 = pl.multiple_of((g >> 7) << 7, 128)
        gl = g - g128

        hs = []
        for c in range(1, 5):
            hs.append(pltpu.async_copy(
                ep_hbm.at[c, pl.ds(rb, 8), pl.ds(g128, 128)],
                ebuf.at[c - 1], sem_f))
        hs.append(pltpu.async_copy(
            line_hbm.at[r, pl.ds(0, 2), pl.ds(g128, 128)], lbuf, sem_f))
        hs.append(pltpu.async_copy(
            perp_hbm.at[r, pl.ds(0, 2), pl.ds(g128, 128)], pbuf, sem_f))
        for h in hs:
            h.wait()

        sub = jnp.full((L,), lane, jnp.int32)
        glv = jnp.full((L,), gl, jnp.int32)
        ev = plsc.load_gather(ebuf, [iota & 3, sub, glv])
        lv = plsc.load_gather(lbuf, [iota & 1, glv])
        pv = plsc.load_gather(pbuf, [iota & 1, glv])
        obuf[...] = jnp.where(iota < 4, ev, jnp.where(iota < 6, lv, pv))
        pltpu.sync_copy(obuf.at[pl.ds(0, 8)],
                        out_hbm.at[pl.ds(pl.multiple_of(r * 8, 8), 8)])


def kernel(endpoints_pred, line_vec, perp_vec):
    ep_t = jnp.transpose(endpoints_pred, (2, 0, 1))
    line_t = jnp.transpose(line_vec, (0, 2, 1))
    perp_t = jnp.transpose(perp_vec, (0, 2, 1))
    out = _select_kernel(ep_t, line_t, perp_t)[0].reshape(B, 8)
    selected_endpoints = out[:, 0:4].reshape(B, 2, 2)
    selected_line_vec = out[:, 4:6]
    selected_perp_vec = out[:, 6:8]
    return (selected_endpoints, selected_line_vec, selected_perp_vec)


# speedup vs baseline: 175.1931x; 1.0807x over previous
import functools

import jax
import jax.numpy as jnp
from jax import lax
from jax.experimental import pallas as pl
from jax.experimental.pallas import tpu as pltpu
from jax.experimental.pallas import tpu_sc as plsc

B = 64
N = 32768
L = 16
NC, NS = 2, 16
QCOLS = N // 4
HCOLS = QCOLS // 2
ITERS = HCOLS // L
BIG_IDX = 1 << 30

_mesh = plsc.VectorSubcoreMesh(core_axis_name="c", subcore_axis_name="s")


@functools.partial(
    pl.kernel,
    out_type=(jax.ShapeDtypeStruct((4 * 128,), jnp.float32),
              jax.ShapeDtypeStruct((2 * 128,), jnp.float32),
              jax.ShapeDtypeStruct((2 * 128,), jnp.float32),
              jax.ShapeDtypeStruct((2 * 16 * 16,), jnp.float32),
              jax.ShapeDtypeStruct((2 * 16 * 16,), jnp.int32)),
    mesh=_mesh,
    scratch_types=[
        pltpu.VMEM((8, HCOLS), jnp.float32),
        pltpu.VMEM((8, HCOLS), jnp.float32),
        pltpu.VMEM((16,), jnp.float32),
        pltpu.VMEM((16,), jnp.int32),
        pltpu.VMEM((64,), jnp.float32),
        pltpu.VMEM((64,), jnp.int32),
        pltpu.VMEM((16, 8, 128), jnp.float32),
        pltpu.VMEM((8, 2, 128), jnp.float32),
        pltpu.VMEM((16,), jnp.float32),
        pltpu.SemaphoreType.DMA,
        pltpu.SemaphoreType.DMA,
        pltpu.SemaphoreType.DMA,
    ],
    compiler_params=pltpu.CompilerParams(needs_layout_passes=False),
)
def _select_kernel(ep_hbm, line_hbm, perp_hbm,
                   oe_hbm, ol_hbm, op_hbm, xval_hbm, xidx_hbm,
                   c0, c1, valbuf, idxbuf, qval, qidx,
                   tbuf, lpbuf, obuf,
                   sem0, sem1, sem_f):
    cid = lax.axis_index("c")
    sid = lax.axis_index("s")
    band = cid * 4 + (sid >> 2)
    k = sid & 3
    rb = pl.multiple_of(band * 8, 8)
    col0 = pl.multiple_of(k * QCOLS, 128)

    iota = lax.iota(jnp.int32, L)

    def row_scan(buf, s_r, cand_base, mx, ai):
        pos0 = cand_base + iota

        def body(i, carry):
            mx, ai = carry
            cv = buf[s_r, pl.ds(i * L, L)]
            m = cv > mx
            mx = jnp.where(m, cv, mx)
            ai = jnp.where(m, pos0 + i * L, ai)
            return mx, ai

        return lax.fori_loop(0, ITERS, body, (mx, ai), unroll=8)

    h0 = pltpu.async_copy(
        ep_hbm.at[0, pl.ds(rb, 8), pl.ds(col0, HCOLS)], c0, sem0)
    h1 = pltpu.async_copy(
        ep_hbm.at[0, pl.ds(rb, 8), pl.ds(col0 + HCOLS, HCOLS)], c1, sem1)

    neg_inf = jnp.full((L,), -jnp.inf, jnp.float32)
    zeros_i = jnp.zeros((L,), jnp.int32)
    accs = [(neg_inf, zeros_i)] * 8

    h0.wait()
    for s_r in range(8):
        accs[s_r] = row_scan(c0, s_r, col0, *accs[s_r])
    h1.wait()
    for s_r in range(8):
        accs[s_r] = row_scan(c1, s_r, col0 + HCOLS, *accs[s_r])

    valv = jnp.zeros((L,), jnp.float32)
    idxv = jnp.zeros((L,), jnp.int32)
    for s_r in range(8):
        mx, ai = accs[s_r]
        row_max = jnp.max(mx)
        g = jnp.min(jnp.where(mx == row_max, ai, jnp.int32(BIG_IDX)))
        valv = jnp.where(iota == s_r, row_max, valv)
        idxv = jnp.where(iota == s_r, g, idxv)
    valbuf[...] = valv
    idxbuf[...] = idxv

    slot = cid * 16 + sid
    pltpu.sync_copy(valbuf,
                    xval_hbm.at[pl.ds(pl.multiple_of(slot * 16, 8), 16)])
    pltpu.sync_copy(idxbuf,
                    xidx_hbm.at[pl.ds(pl.multiple_of(slot * 16, 8), 16)])
    plsc.subcore_barrier()
    bslot = cid * 16 + (sid & ~3)
    pltpu.sync_copy(xval_hbm.at[pl.ds(pl.multiple_of(bslot * 16, 8), 64)],
                    qval)
    pltpu.sync_copy(xidx_hbm.at[pl.ds(pl.multiple_of(bslot * 16, 8), 64)],
                    qidx)

    mv = qval[pl.ds(0, L)]
    mi = qidx[pl.ds(0, L)]
    for q in range(1, 4):
        v = qval[pl.ds(q * 16, L)]
        i_ = qidx[pl.ds(q * 16, L)]
        m = v > mv
        mv = jnp.where(m, v, mv)
        mi = jnp.where(m, i_, mi)

    mi = jnp.clip(mi, 0, N - 1)
    glv = mi & 127
    gbases = []
    for l in range(8):
        g_l = jnp.max(jnp.where(iota == l, mi, jnp.int32(0)))
        gbases.append(pl.multiple_of((g_l >> 7) << 7, 128))

    @pl.when(k < 2)
    def _():
        cbase = 1 + k * 2
        hs = []
        for dc in range(2):
            for l in range(8):
                hs.append(pltpu.async_copy(
                    ep_hbm.at[cbase + dc, pl.ds(rb, 8), pl.ds(gbases[l], 128)],
                    tbuf.at[dc * 8 + l], sem_f))
        for h in hs:
            h.wait()
        for dc in range(2):
            vals = plsc.load_gather(
                tbuf, [jnp.int32(dc * 8) + (iota & 7), iota & 7, glv])
            obuf[...] = vals
            comp = cbase + dc - 1
            pltpu.sync_copy(
                obuf.at[pl.ds(0, 8)],
                oe_hbm.at[pl.ds(pl.multiple_of(comp * 128 + rb, 8), 8)])

    for kk, (src_hbm, dst_hbm) in ((2, (line_hbm, ol_hbm)),
                                   (3, (perp_hbm, op_hbm))):
        @pl.when(k == kk)
        def _(src_hbm=src_hbm, dst_hbm=dst_hbm):
            hs = []
            for l in range(8):
                hs.append(pltpu.async_copy(
                    src_hbm.at[rb + l, pl.ds(0, 2), pl.ds(gbases[l], 128)],
                    lpbuf.at[l], sem_f))
            for h in hs:
                h.wait()
            for j in range(2):
                vals = plsc.load_gather(
                    lpbuf, [iota & 7, jnp.full((L,), j, jnp.int32), glv])
                obuf[...] = vals
                pltpu.sync_copy(
                    obuf.at[pl.ds(0, 8)],
                    dst_hbm.at[pl.ds(pl.multiple_of(j * 128 + rb, 8), 8)])


def kernel(endpoints_pred, line_vec, perp_vec):
    ep_t = jnp.transpose(endpoints_pred, (2, 0, 1))
    line_t = jnp.transpose(line_vec, (0, 2, 1))
    perp_t = jnp.transpose(perp_vec, (0, 2, 1))
    oe, ol, op = _select_kernel(ep_t, line_t, perp_t)[:3]
    selected_endpoints = jnp.transpose(oe.reshape(2, 2, 128), (2, 0, 1))[:B]
    selected_line_vec = jnp.transpose(ol.reshape(2, 128), (1, 0))[:B]
    selected_perp_vec = jnp.transpose(op.reshape(2, 128), (1, 0))[:B]
    return (selected_endpoints, selected_line_vec, selected_perp_vec)


# speedup vs baseline: 194.4056x; 1.1097x over previous
import functools

import jax
import jax.numpy as jnp
from jax import lax
from jax.experimental import pallas as pl
from jax.experimental.pallas import tpu as pltpu
from jax.experimental.pallas import tpu_sc as plsc

B = 64
N = 32768
L = 16
NC, NS = 2, 16
QCOLS = N // 4
HCOLS = QCOLS // 2
ITERS = HCOLS // L
BIG_IDX = 1 << 30

_mesh = plsc.VectorSubcoreMesh(core_axis_name="c", subcore_axis_name="s")


@functools.partial(
    pl.kernel,
    out_type=(jax.ShapeDtypeStruct((4 * 128,), jnp.float32),
              jax.ShapeDtypeStruct((2 * 128,), jnp.float32),
              jax.ShapeDtypeStruct((2 * 128,), jnp.float32),
              jax.ShapeDtypeStruct((2 * 16 * 16,), jnp.float32),
              jax.ShapeDtypeStruct((2 * 16 * 16,), jnp.int32)),
    mesh=_mesh,
    scratch_types=[
        pltpu.VMEM((8, HCOLS), jnp.float32),
        pltpu.VMEM((8, HCOLS), jnp.float32),
        pltpu.VMEM((16,), jnp.float32),
        pltpu.VMEM((16,), jnp.int32),
        pltpu.VMEM((64,), jnp.float32),
        pltpu.VMEM((64,), jnp.int32),
        pltpu.VMEM((16, 8, 128), jnp.float32),
        pltpu.VMEM((8, 2, 128), jnp.float32),
        pltpu.VMEM((16,), jnp.float32),
        pltpu.SemaphoreType.DMA,
        pltpu.SemaphoreType.DMA,
        pltpu.SemaphoreType.DMA,
    ],
    compiler_params=pltpu.CompilerParams(needs_layout_passes=False),
)
def _select_kernel(ep_hbm, line_hbm, perp_hbm,
                   oe_hbm, ol_hbm, op_hbm, xval_hbm, xidx_hbm,
                   c0, c1, valbuf, idxbuf, qval, qidx,
                   tbuf, lpbuf, obuf,
                   sem0, sem1, sem_f):
    cid = lax.axis_index("c")
    sid = lax.axis_index("s")
    band = cid * 4 + (sid >> 2)
    k = sid & 3
    rb = pl.multiple_of(band * 8, 8)
    col0 = pl.multiple_of(k * QCOLS, 128)

    iota = lax.iota(jnp.int32, L)

    def block_scan(buf, cand_base, accs):
        pos0 = cand_base + iota

        def body(i, accs):
            pos = pos0 + i * L
            out = []
            for s_r in range(8):
                mx, ai = accs[s_r]
                cv = buf[s_r, pl.ds(i * L, L)]
                m = cv > mx
                out.append((jnp.where(m, cv, mx), jnp.where(m, pos, ai)))
            return tuple(out)

        return lax.fori_loop(0, ITERS, body, tuple(accs), unroll=2)

    h0 = pltpu.async_copy(
        ep_hbm.at[0, pl.ds(rb, 8), pl.ds(col0, HCOLS)], c0, sem0)
    h1 = pltpu.async_copy(
        ep_hbm.at[0, pl.ds(rb, 8), pl.ds(col0 + HCOLS, HCOLS)], c1, sem1)

    neg_inf = jnp.full((L,), -jnp.inf, jnp.float32)
    zeros_i = jnp.zeros((L,), jnp.int32)
    accs = [(neg_inf, zeros_i)] * 8

    h0.wait()
    accs = block_scan(c0, col0, accs)
    h1.wait()
    accs = block_scan(c1, col0 + HCOLS, accs)

    valv = jnp.zeros((L,), jnp.float32)
    idxv = jnp.zeros((L,), jnp.int32)
    for s_r in range(8):
        mx, ai = accs[s_r]
        row_max = jnp.max(mx)
        g = jnp.min(jnp.where(mx == row_max, ai, jnp.int32(BIG_IDX)))
        valv = jnp.where(iota == s_r, row_max, valv)
        idxv = jnp.where(iota == s_r, g, idxv)
    valbuf[...] = valv
    idxbuf[...] = idxv

    slot = cid * 16 + sid
    pltpu.sync_copy(valbuf,
                    xval_hbm.at[pl.ds(pl.multiple_of(slot * 16, 8), 16)])
    pltpu.sync_copy(idxbuf,
                    xidx_hbm.at[pl.ds(pl.multiple_of(slot * 16, 8), 16)])
    plsc.subcore_barrier()
    bslot = cid * 16 + (sid & ~3)
    pltpu.sync_copy(xval_hbm.at[pl.ds(pl.multiple_of(bslot * 16, 8), 64)],
                    qval)
    pltpu.sync_copy(xidx_hbm.at[pl.ds(pl.multiple_of(bslot * 16, 8), 64)],
                    qidx)

    mv = qval[pl.ds(0, L)]
    mi = qidx[pl.ds(0, L)]
    for q in range(1, 4):
        v = qval[pl.ds(q * 16, L)]
        i_ = qidx[pl.ds(q * 16, L)]
        m = v > mv
        mv = jnp.where(m, v, mv)
        mi = jnp.where(m, i_, mi)

    mi = jnp.clip(mi, 0, N - 1)
    glv = mi & 127
    gbases = []
    for l in range(8):
        g_l = jnp.max(jnp.where(iota == l, mi, jnp.int32(0)))
        gbases.append(pl.multiple_of((g_l >> 7) << 7, 128))

    @pl.when(k < 2)
    def _():
        cbase = 1 + k * 2
        hs = []
        for dc in range(2):
            for l in range(8):
                hs.append(pltpu.async_copy(
                    ep_hbm.at[cbase + dc, pl.ds(rb, 8), pl.ds(gbases[l], 128)],
                    tbuf.at[dc * 8 + l], sem_f))
        for h in hs:
            h.wait()
        for dc in range(2):
            vals = plsc.load_gather(
                tbuf, [jnp.int32(dc * 8) + (iota & 7), iota & 7, glv])
            obuf[...] = vals
            comp = cbase + dc - 1
            pltpu.sync_copy(
                obuf.at[pl.ds(0, 8)],
                oe_hbm.at[pl.ds(pl.multiple_of(comp * 128 + rb, 8), 8)])

    for kk, (src_hbm, dst_hbm) in ((2, (line_hbm, ol_hbm)),
                                   (3, (perp_hbm, op_hbm))):
        @pl.when(k == kk)
        def _(src_hbm=src_hbm, dst_hbm=dst_hbm):
            hs = []
            for l in range(8):
                hs.append(pltpu.async_copy(
                    src_hbm.at[rb + l, pl.ds(0, 2), pl.ds(gbases[l], 128)],
                    lpbuf.at[l], sem_f))
            for h in hs:
                h.wait()
            for j in range(2):
                vals = plsc.load_gather(
                    lpbuf, [iota & 7, jnp.full((L,), j, jnp.int32), glv])
                obuf[...] = vals
                pltpu.sync_copy(
                    obuf.at[pl.ds(0, 8)],
                    dst_hbm.at[pl.ds(pl.multiple_of(j * 128 + rb, 8), 8)])


def kernel(endpoints_pred, line_vec, perp_vec):
    ep_t = jnp.transpose(endpoints_pred, (2, 0, 1))
    line_t = jnp.transpose(line_vec, (0, 2, 1))
    perp_t = jnp.transpose(perp_vec, (0, 2, 1))
    oe, ol, op = _select_kernel(ep_t, line_t, perp_t)[:3]
    selected_endpoints = jnp.transpose(oe.reshape(2, 2, 128), (2, 0, 1))[:B]
    selected_line_vec = jnp.transpose(ol.reshape(2, 128), (1, 0))[:B]
    selected_perp_vec = jnp.transpose(op.reshape(2, 128), (1, 0))[:B]
    return (selected_endpoints, selected_line_vec, selected_perp_vec)


# speedup vs baseline: 199.9083x; 1.0283x over previous
import functools

import jax
import jax.numpy as jnp
from jax import lax
from jax.experimental import pallas as pl
from jax.experimental.pallas import tpu as pltpu
from jax.experimental.pallas import tpu_sc as plsc

B = 64
N = 32768
L = 16
NC, NS = 2, 16
QCOLS = N // 4
NCH = 4
CCOLS = QCOLS // NCH
ITERS = CCOLS // L
BIG_IDX = 1 << 30

_mesh = plsc.VectorSubcoreMesh(core_axis_name="c", subcore_axis_name="s")


@functools.partial(
    pl.kernel,
    out_type=(jax.ShapeDtypeStruct((4 * 128,), jnp.float32),
              jax.ShapeDtypeStruct((2 * 128,), jnp.float32),
              jax.ShapeDtypeStruct((2 * 128,), jnp.float32),
              jax.ShapeDtypeStruct((2 * 16 * 32,), jnp.float32)),
    mesh=_mesh,
    scratch_types=[
        pltpu.VMEM((8, CCOLS), jnp.float32),
        pltpu.VMEM((8, CCOLS), jnp.float32),
        pltpu.VMEM((32,), jnp.float32),
        pltpu.VMEM((128,), jnp.float32),
        pltpu.VMEM((16, 8, 128), jnp.float32),
        pltpu.VMEM((8, 2, 128), jnp.float32),
        pltpu.VMEM((16,), jnp.float32),
        pltpu.SemaphoreType.DMA,
        pltpu.SemaphoreType.DMA,
        pltpu.SemaphoreType.DMA,
    ],
    compiler_params=pltpu.CompilerParams(needs_layout_passes=False),
)
def _select_kernel(ep_hbm, line_hbm, perp_hbm,
                   oe_hbm, ol_hbm, op_hbm, xpk_hbm,
                   c0, c1, pkbuf, qpk,
                   tbuf, lpbuf, obuf,
                   sem0, sem1, sem_f):
    cid = lax.axis_index("c")
    sid = lax.axis_index("s")
    band = cid * 4 + (sid >> 2)
    k = sid & 3
    rb = pl.multiple_of(band * 8, 8)
    col0 = pl.multiple_of(k * QCOLS, 128)

    iota = lax.iota(jnp.int32, L)

    def block_scan(buf, cand_base, accs):
        pos0 = cand_base + iota

        def body(i, accs):
            pos = pos0 + i * L
            out = []
            for s_r in range(8):
                mx, ai = accs[s_r]
                cv = buf[s_r, pl.ds(i * L, L)]
                m = cv > mx
                out.append((jnp.where(m, cv, mx), jnp.where(m, pos, ai)))
            return tuple(out)

        return lax.fori_loop(0, ITERS, body, tuple(accs), unroll=2)

    bufs = (c0, c1)
    sems = (sem0, sem1)
    hs = [None, None]
    for ci in range(2):
        hs[ci] = pltpu.async_copy(
            ep_hbm.at[0, pl.ds(rb, 8), pl.ds(col0 + ci * CCOLS, CCOLS)],
            bufs[ci], sems[ci])

    neg_inf = jnp.full((L,), -jnp.inf, jnp.float32)
    zeros_i = jnp.zeros((L,), jnp.int32)
    accs = [(neg_inf, zeros_i)] * 8

    for ci in range(NCH):
        par = ci & 1
        hs[par].wait()
        accs = block_scan(bufs[par], col0 + ci * CCOLS, accs)
        if ci + 2 < NCH:
            hs[par] = pltpu.async_copy(
                ep_hbm.at[0, pl.ds(rb, 8),
                          pl.ds(col0 + (ci + 2) * CCOLS, CCOLS)],
                bufs[par], sems[par])

    valv = jnp.zeros((L,), jnp.float32)
    idxv = jnp.zeros((L,), jnp.int32)
    for s_r in range(8):
        mx, ai = accs[s_r]
        row_max = jnp.max(mx)
        g = jnp.min(jnp.where(mx == row_max, ai, jnp.int32(BIG_IDX)))
        valv = jnp.where(iota == s_r, row_max, valv)
        idxv = jnp.where(iota == s_r, g, idxv)
    pkbuf[pl.ds(0, L)] = valv
    pkbuf[pl.ds(16, L)] = plsc.bitcast(idxv, jnp.float32)

    slot = cid * 16 + sid
    pltpu.sync_copy(pkbuf,
                    xpk_hbm.at[pl.ds(pl.multiple_of(slot * 32, 8), 32)])
    plsc.subcore_barrier()
    bslot = cid * 16 + (sid & ~3)
    pltpu.sync_copy(xpk_hbm.at[pl.ds(pl.multiple_of(bslot * 32, 8), 128)],
                    qpk)

    mv = qpk[pl.ds(0, L)]
    mi = plsc.bitcast(qpk[pl.ds(16, L)], jnp.int32)
    for q in range(1, 4):
        v = qpk[pl.ds(q * 32, L)]
        i_ = plsc.bitcast(qpk[pl.ds(q * 32 + 16, L)], jnp.int32)
        m = v > mv
        mv = jnp.where(m, v, mv)
        mi = jnp.where(m, i_, mi)

    mi = jnp.clip(mi, 0, N - 1)
    glv = mi & 127
    gbases = []
    for l in range(8):
        g_l = jnp.max(jnp.where(iota == l, mi, jnp.int32(0)))
        gbases.append(pl.multiple_of((g_l >> 7) << 7, 128))

    @pl.when(k < 2)
    def _():
        cbase = 1 + k * 2
        hs = []
        for dc in range(2):
            for l in range(8):
                hs.append(pltpu.async_copy(
                    ep_hbm.at[cbase + dc, pl.ds(rb, 8), pl.ds(gbases[l], 128)],
                    tbuf.at[dc * 8 + l], sem_f))
        for h in hs:
            h.wait()
        for dc in range(2):
            vals = plsc.load_gather(
                tbuf, [jnp.int32(dc * 8) + (iota & 7), iota & 7, glv])
            obuf[...] = vals
            comp = cbase + dc - 1
            pltpu.sync_copy(
                obuf.at[pl.ds(0, 8)],
                oe_hbm.at[pl.ds(pl.multiple_of(comp * 128 + rb, 8), 8)])

    for kk, (src_hbm, dst_hbm) in ((2, (line_hbm, ol_hbm)),
                                   (3, (perp_hbm, op_hbm))):
        @pl.when(k == kk)
        def _(src_hbm=src_hbm, dst_hbm=dst_hbm):
            hs = []
            for l in range(8):
                hs.append(pltpu.async_copy(
                    src_hbm.at[rb + l, pl.ds(0, 2), pl.ds(gbases[l], 128)],
                    lpbuf.at[l], sem_f))
            for h in hs:
                h.wait()
            for j in range(2):
                vals = plsc.load_gather(
                    lpbuf, [iota & 7, jnp.full((L,), j, jnp.int32), glv])
                obuf[...] = vals
                pltpu.sync_copy(
                    obuf.at[pl.ds(0, 8)],
                    dst_hbm.at[pl.ds(pl.multiple_of(j * 128 + rb, 8), 8)])


def kernel(endpoints_pred, line_vec, perp_vec):
    ep_t = jnp.transpose(endpoints_pred, (2, 0, 1))
    line_t = jnp.transpose(line_vec, (0, 2, 1))
    perp_t = jnp.transpose(perp_vec, (0, 2, 1))
    oe, ol, op = _select_kernel(ep_t, line_t, perp_t)[:3]
    selected_endpoints = jnp.transpose(oe.reshape(2, 2, 128), (2, 0, 1))[:B]
    selected_line_vec = jnp.transpose(ol.reshape(2, 128), (1, 0))[:B]
    selected_perp_vec = jnp.transpose(op.reshape(2, 128), (1, 0))[:B]
    return (selected_endpoints, selected_line_vec, selected_perp_vec)
